# Initial kernel scaffold; baseline (speedup 1.0000x reference)
import jax
import jax.numpy as jnp
from jax.experimental import pallas as pl


def kernel(ids, table, W, b):
    raise NotImplementedError("write your pallas kernel here")


# speedup vs baseline: 1.9663x; 1.9663x over previous
import jax
import jax.numpy as jnp
from jax import lax
from jax.experimental import pallas as pl
from jax.experimental.pallas import tpu as pltpu
from jax.experimental.pallas import tpu_sc as plsc

B = 4096
L = 200
DIM = 32
NW = 32
RB = B // NW
C0 = 104
C1 = L - C0


def _sc_pool_body(ids_hbm, table_hbm, out_hbm, ids_v, buf0, buf1, sums_v,
                  sem0, sem1):
    wid = lax.axis_index("s") * 2 + lax.axis_index("c")
    base = wid * RB

    pltpu.sync_copy(ids_hbm.at[pl.ds(base * L, RB * L)], ids_v)

    def row_body(b, carry):
        off = b * L
        g0 = pltpu.async_copy(table_hbm.at[ids_v.at[pl.ds(off, C0)]],
                              buf0, sem0)
        g1 = pltpu.async_copy(table_hbm.at[ids_v.at[pl.ds(off + C0, C1)]],
                              buf1, sem1)
        g0.wait()
        g1.wait()

        a0 = jnp.zeros((16,), jnp.float32)
        a1 = jnp.zeros((16,), jnp.float32)
        a2 = jnp.zeros((16,), jnp.float32)
        a3 = jnp.zeros((16,), jnp.float32)
        for r in range(0, C0, 2):
            a0 = a0 + buf0[r, pl.ds(0, 16)]
            a1 = a1 + buf0[r, pl.ds(16, 16)]
            a2 = a2 + buf0[r + 1, pl.ds(0, 16)]
            a3 = a3 + buf0[r + 1, pl.ds(16, 16)]
        for r in range(0, C1, 2):
            a0 = a0 + buf1[r, pl.ds(0, 16)]
            a1 = a1 + buf1[r, pl.ds(16, 16)]
            a2 = a2 + buf1[r + 1, pl.ds(0, 16)]
            a3 = a3 + buf1[r + 1, pl.ds(16, 16)]

        sums_v[b, pl.ds(0, 16)] = a0 + a2
        sums_v[b, pl.ds(16, 16)] = a1 + a3
        return carry

    lax.fori_loop(0, RB, row_body, 0)

    pltpu.sync_copy(sums_v, out_hbm.at[pl.ds(base, RB)])


@jax.jit
def _sc_pool(ids_flat, table):
    mesh = plsc.VectorSubcoreMesh(core_axis_name="c", subcore_axis_name="s")
    return pl.kernel(
        _sc_pool_body,
        out_type=jax.ShapeDtypeStruct((B, DIM), jnp.float32),
        mesh=mesh,
        compiler_params=pltpu.CompilerParams(use_tc_tiling_on_sc=False),
        scratch_types=[
            pltpu.VMEM((RB * L,), jnp.int32),
            pltpu.VMEM((C0, DIM), jnp.float32),
            pltpu.VMEM((C1, DIM), jnp.float32),
            pltpu.VMEM((RB, DIM), jnp.float32),
            pltpu.SemaphoreType.DMA,
            pltpu.SemaphoreType.DMA,
        ],
    )(ids_flat, table)


def _tc_body(ids_ref, s_ref, w_ref, b_ref, o_ref):
    cnt = jnp.sum((ids_ref[...] != 0).astype(jnp.float32), axis=1,
                  keepdims=True)
    pooled = s_ref[...] * (1.0 / jnp.maximum(cnt, 1.0))
    o_ref[...] = (
        jnp.dot(pooled, w_ref[...].T, preferred_element_type=jnp.float32)
        + b_ref[...]
    )


@jax.jit
def _tc_head(ids, sums, W, b):
    return pl.pallas_call(
        _tc_body,
        out_shape=jax.ShapeDtypeStruct((B, W.shape[0]), jnp.float32),
    )(ids, sums, W, b.reshape(1, -1))


def kernel(ids, table, W, b):
    ids_flat = ids.reshape(-1).astype(jnp.int32)
    sums = _sc_pool(ids_flat, table)
    return _tc_head(ids, sums, W, b)
